# speedup vs baseline: 8.9757x; 8.9757x over previous
import functools
import math

import jax
import jax.numpy as jnp
import numpy as np
from jax import lax
from jax.experimental import pallas as pl
from jax.experimental.pallas import tpu as pltpu

_INTERPRET = False

H = 32
WMAP = (0, 1, 1, 1, 2, 2, 2, 2, 2)


def _mm(A, B):
    return [sum(A[3 * i + j] * B[3 * j + k] for j in range(3))
            for i in range(3) for k in range(3)]


def _decomp(t):
    m = (t[0] + t[4] + t[8]) * (1.0 / 3.0)
    return (m,
            0.5 * (t[1] - t[3]), 0.5 * (t[2] - t[6]), 0.5 * (t[5] - t[7]),
            t[0] - m, 0.5 * (t[1] + t[3]), 0.5 * (t[2] + t[6]),
            t[4] - m, 0.5 * (t[5] + t[7]))


def _edge_body(attr, w, w0t, b0, w1t, b1, out):
    ea = jnp.tanh(jnp.dot(attr[...], w0t[...],
                          preferred_element_type=jnp.float32) + b0[...])
    ea = jnp.tanh(jnp.dot(ea, w1t[...],
                          preferred_element_type=jnp.float32) + b1[...])
    wv = w[...]
    cut = 0.5 * (jnp.cos(wv * (math.pi / 5.0)) + 1.0) * (wv < 5.0)
    ea = ea * cut
    out[0] = ea[:, :48]
    out[1] = ea[:, 48:]


def _edge_mlp(edge_attr, edge_weight, lin0_W, lin0_b, lin1_W, lin1_b):
    E = edge_attr.shape[0]
    EB = 3200
    grid = E // EB
    return pl.pallas_call(
        _edge_body,
        grid=(grid,),
        in_specs=[
            pl.BlockSpec((EB, 32), lambda i: (i, 0)),
            pl.BlockSpec((EB, 1), lambda i: (i, 0)),
            pl.BlockSpec((32, 96), lambda i: (0, 0)),
            pl.BlockSpec((1, 96), lambda i: (0, 0)),
            pl.BlockSpec((96, 96), lambda i: (0, 0)),
            pl.BlockSpec((1, 96), lambda i: (0, 0)),
        ],
        out_specs=pl.BlockSpec((2, EB, 48), lambda i: (0, i, 0)),
        out_shape=jax.ShapeDtypeStruct((2, E, 48), jnp.float32),
        interpret=_INTERPRET,
    )(edge_attr, edge_weight.reshape(E, 1), lin0_W.T,
      lin0_b.reshape(1, 96), lin1_W.T, lin1_b.reshape(1, 96))


def _prep_body(x, w2t, w3t, w4t, yfull, ypacked):
    xs = [x[p] for p in range(9)]
    m, a01, a02, a12, s00, s01, s02, s11, s12 = _decomp(xs)
    dot = lambda a, wt: jnp.dot(a, wt[...], preferred_element_type=jnp.float32)
    mY = dot(m, w4t)
    aY01 = dot(a01, w3t); aY02 = dot(a02, w3t); aY12 = dot(a12, w3t)
    sY00 = dot(s00, w2t); sY01 = dot(s01, w2t); sY02 = dot(s02, w2t)
    sY11 = dot(s11, w2t); sY12 = dot(s12, w2t)
    comps = [mY, aY01, aY02, aY12, sY00, sY01, sY02, sY11, sY12]
    yf = [mY + sY00, aY01 + sY01, aY02 + sY02,
          -aY01 + sY01, mY + sY11, aY12 + sY12,
          -aY02 + sY02, -aY12 + sY12, mY - sY00 - sY11]
    for p in range(9):
        yfull[p] = yf[p]
    for s in range(2):
        ypacked[s] = jnp.concatenate(
            [c[:, s * 16:(s + 1) * 16] for c in comps], axis=1)


def _node_prep(X9, lin2_W, lin3_W, lin4_W):
    Np = X9.shape[1]
    NB = 1000
    grid = Np // NB
    return pl.pallas_call(
        _prep_body,
        grid=(grid,),
        in_specs=[
            pl.BlockSpec((9, NB, H), lambda i: (0, i, 0)),
            pl.BlockSpec((H, H), lambda i: (0, 0)),
            pl.BlockSpec((H, H), lambda i: (0, 0)),
            pl.BlockSpec((H, H), lambda i: (0, 0)),
        ],
        out_specs=[
            pl.BlockSpec((9, NB, H), lambda i: (0, i, 0)),
            pl.BlockSpec((2, NB, 144), lambda i: (0, i, 0)),
        ],
        out_shape=[
            jax.ShapeDtypeStruct((9, Np, H), jnp.float32),
            jax.ShapeDtypeStruct((2, Np, 144), jnp.float32),
        ],
        interpret=_INTERPRET,
    )(X9, lin2_W.T, lin3_W.T, lin4_W.T)


def _final_body(x, yfull, mp, w5t, w6t, w7t, t0t, t1t, t2t,
                ls0t, ls0b, ls1pt, ls1pb, out):
    xs = [x[p] for p in range(9)]
    yf = [yfull[p] for p in range(9)]
    macc = [jnp.concatenate([mp[0, :, c * 16:(c + 1) * 16],
                             mp[1, :, c * 16:(c + 1) * 16]], axis=1)
            for c in range(9)]
    M = [macc[0] + macc[4], macc[1] + macc[5], macc[2] + macc[6],
         -macc[1] + macc[5], macc[0] + macc[7], macc[3] + macc[8],
         -macc[2] + macc[6], -macc[3] + macc[8], macc[0] - macc[4] - macc[7]]
    Z = _mm(M, _mm(yf, M))
    mZ, za01, za02, za12, zs00, zs01, zs02, zs11, zs12 = _decomp(Z)
    dot = lambda a, wt: jnp.dot(a, wt[...], preferred_element_type=jnp.float32)
    mW = dot(mZ, w7t)
    wa01 = dot(za01, w6t); wa02 = dot(za02, w6t); wa12 = dot(za12, w6t)
    ws00 = dot(zs00, w5t); ws01 = dot(zs01, w5t); ws02 = dot(zs02, w5t)
    ws11 = dot(zs11, w5t); ws12 = dot(zs12, w5t)
    Y2 = [xs[0] + mW + ws00, xs[1] + wa01 + ws01, xs[2] + wa02 + ws02,
          xs[3] - wa01 + ws01, xs[4] + mW + ws11, xs[5] + wa12 + ws12,
          xs[6] - wa02 + ws02, xs[7] - wa12 + ws12, xs[8] + mW - ws00 - ws11]
    tn = sum(p * p for p in Y2) + 1.0
    inv = 1.0 / tn
    Y2 = [p * inv for p in Y2]
    m3, b01, b02, b12, c00, c01, c02, c11, c12 = _decomp(Y2)
    tnI = 3.0 * m3 * m3
    tnA = 2.0 * (b01 * b01 + b02 * b02 + b12 * b12)
    c22 = -c00 - c11
    tnS = (c00 * c00 + c11 * c11 + c22 * c22
           + 2.0 * (c01 * c01 + c02 * c02 + c12 * c12))
    nrm = jnp.concatenate([tnI, tnA, tnS], axis=1)
    nrm = jnp.tanh(dot(nrm, ls0t) + ls0b[...])
    nrm = jnp.tanh(dot(nrm, ls1pt) + ls1pb[...])
    f0 = nrm[:, :H]; f1 = nrm[:, H:2 * H]; f2 = nrm[:, 2 * H:]
    dm = dot(m3, t0t)
    da01 = dot(b01, t1t); da02 = dot(b02, t1t); da12 = dot(b12, t1t)
    ds00 = dot(c00, t2t); ds01 = dot(c01, t2t); ds02 = dot(c02, t2t)
    ds11 = dot(c11, t2t); ds12 = dot(c12, t2t)
    D = [f0 * dm + f2 * ds00, f1 * da01 + f2 * ds01, f1 * da02 + f2 * ds02,
         -f1 * da01 + f2 * ds01, f0 * dm + f2 * ds11, f1 * da12 + f2 * ds12,
         -f1 * da02 + f2 * ds02, -f1 * da12 + f2 * ds12,
         f0 * dm - f2 * (ds00 + ds11)]
    DD = _mm(D, D)
    for p in range(9):
        out[p] = xs[p] + D[p] - 0.5 * DD[p]


def _final(X9, yfull, msgpacked, lin5_W, lin6_W, lin7_W, lt0_W, lt1_W, lt2_W,
           ls0_W, ls0_b, ls1_Wp, ls1_bp):
    Np = X9.shape[1]
    NB = 1000
    grid = Np // NB
    return pl.pallas_call(
        _final_body,
        grid=(grid,),
        in_specs=[
            pl.BlockSpec((9, NB, H), lambda i: (0, i, 0)),
            pl.BlockSpec((9, NB, H), lambda i: (0, i, 0)),
            pl.BlockSpec((2, NB, 144), lambda i: (0, i, 0)),
            pl.BlockSpec((H, H), lambda i: (0, 0)),
            pl.BlockSpec((H, H), lambda i: (0, 0)),
            pl.BlockSpec((H, H), lambda i: (0, 0)),
            pl.BlockSpec((H, H), lambda i: (0, 0)),
            pl.BlockSpec((H, H), lambda i: (0, 0)),
            pl.BlockSpec((H, H), lambda i: (0, 0)),
            pl.BlockSpec((96, 96), lambda i: (0, 0)),
            pl.BlockSpec((1, 96), lambda i: (0, 0)),
            pl.BlockSpec((96, 96), lambda i: (0, 0)),
            pl.BlockSpec((1, 96), lambda i: (0, 0)),
        ],
        out_specs=pl.BlockSpec((9, NB, H), lambda i: (0, i, 0)),
        out_shape=jax.ShapeDtypeStruct((9, Np, H), jnp.float32),
        interpret=_INTERPRET,
    )(X9, yfull, msgpacked, lin5_W.T, lin6_W.T, lin7_W.T,
      lt0_W.T, lt1_W.T, lt2_W.T, ls0_W.T, ls0_b.reshape(1, 96),
      ls1_Wp.T, ls1_bp.reshape(1, 96))


def _message_pass(ypacked, ea48, src, dst, Np, E):
    emap = np.empty((144,), np.int32)
    for c in range(9):
        for h16 in range(16):
            emap[c * 16 + h16] = h16 * 3 + WMAP[c]
    out = []
    for s in range(2):
        wexp = ea48[s][:, emap]
        rows = ypacked[s][src]
        out.append(jnp.zeros((Np, 144), jnp.float32).at[dst].add(wexp * rows))
    return jnp.stack(out, 0)


def kernel(X, edge_index, edge_weight, edge_attr, lin0_W, lin0_b, lin1_W,
           lin1_b, lin2_W, lin3_W, lin4_W, lin5_W, lin6_W, lin7_W, lt0_W,
           lt1_W, lt2_W, ls0_W, ls0_b, ls1_W, ls1_b):
    Np = X.shape[0]
    E = edge_index.shape[1]
    X9 = jnp.transpose(X.reshape(Np, H, 9), (2, 0, 1))

    ea48 = _edge_mlp(edge_attr, edge_weight, lin0_W, lin0_b, lin1_W, lin1_b)
    yfull, ypacked = _node_prep(X9, lin2_W, lin3_W, lin4_W)

    src = edge_index[0].astype(jnp.int32)
    dst = edge_index[1].astype(jnp.int32)
    msgpacked = _message_pass(ypacked, ea48, src, dst, Np, E)

    perm = np.array([3 * h + k for k in range(3) for h in range(H)])
    out9 = _final(X9, yfull, msgpacked, lin5_W, lin6_W, lin7_W,
                  lt0_W, lt1_W, lt2_W, ls0_W, ls0_b, ls1_W[perm], ls1_b[perm])
    return jnp.transpose(out9, (1, 2, 0)).reshape(Np, H, 3, 3)


# speedup vs baseline: 20.8888x; 2.3273x over previous
import functools
import math

import jax
import jax.numpy as jnp
import numpy as np
from jax import lax
from jax.experimental import pallas as pl
from jax.experimental.pallas import tpu as pltpu

_INTERPRET = False

_EPERM = np.array([(16 * s + h16) * 3 + k
                   for s in range(2) for k in range(3) for h16 in range(16)])

H = 32
WMAP = (0, 1, 1, 1, 2, 2, 2, 2, 2)


def _mm(A, B):
    return [sum(A[3 * i + j] * B[3 * j + k] for j in range(3))
            for i in range(3) for k in range(3)]


def _decomp(t):
    m = (t[0] + t[4] + t[8]) * (1.0 / 3.0)
    return (m,
            0.5 * (t[1] - t[3]), 0.5 * (t[2] - t[6]), 0.5 * (t[5] - t[7]),
            t[0] - m, 0.5 * (t[1] + t[3]), 0.5 * (t[2] + t[6]),
            t[4] - m, 0.5 * (t[5] + t[7]))


def _edge_body(attr, w, w0t, b0, w1t, b1, out):
    ea = jnp.tanh(jnp.dot(attr[...], w0t[...],
                          preferred_element_type=jnp.float32) + b0[...])
    ea = jnp.tanh(jnp.dot(ea, w1t[...],
                          preferred_element_type=jnp.float32) + b1[...])
    wv = w[...]
    cut = 0.5 * (jnp.cos(wv * (math.pi / 5.0)) + 1.0) * (wv < 5.0)
    ea = ea * cut
    out[0] = ea[:, :48]
    out[1] = ea[:, 48:]


def _edge_mlp(edge_attr, edge_weight, lin0_W, lin0_b, lin1_W, lin1_b):
    E = edge_attr.shape[0]
    EB = next(eb for eb in range(4096, 0, -8) if E % eb == 0)
    grid = E // EB
    return pl.pallas_call(
        _edge_body,
        grid=(grid,),
        in_specs=[
            pl.BlockSpec((EB, 32), lambda i: (i, 0)),
            pl.BlockSpec((EB, 1), lambda i: (i, 0)),
            pl.BlockSpec((32, 96), lambda i: (0, 0)),
            pl.BlockSpec((1, 96), lambda i: (0, 0)),
            pl.BlockSpec((96, 96), lambda i: (0, 0)),
            pl.BlockSpec((1, 96), lambda i: (0, 0)),
        ],
        out_specs=pl.BlockSpec((2, EB, 48), lambda i: (0, i, 0)),
        out_shape=jax.ShapeDtypeStruct((2, E, 48), jnp.float32),
        interpret=_INTERPRET,
    )(edge_attr, edge_weight.reshape(E, 1), lin0_W.T,
      lin0_b.reshape(1, 96), lin1_W.T[:, _EPERM], lin1_b[_EPERM].reshape(1, 96))


def _prep_body(x, w2t, w3t, w4t, yfull, ypacked):
    xs = [x[p] for p in range(9)]
    m, a01, a02, a12, s00, s01, s02, s11, s12 = _decomp(xs)
    dot = lambda a, wt: jnp.dot(a, wt[...], preferred_element_type=jnp.float32)
    mY = dot(m, w4t)
    aY01 = dot(a01, w3t); aY02 = dot(a02, w3t); aY12 = dot(a12, w3t)
    sY00 = dot(s00, w2t); sY01 = dot(s01, w2t); sY02 = dot(s02, w2t)
    sY11 = dot(s11, w2t); sY12 = dot(s12, w2t)
    comps = [mY, aY01, aY02, aY12, sY00, sY01, sY02, sY11, sY12]
    yf = [mY + sY00, aY01 + sY01, aY02 + sY02,
          -aY01 + sY01, mY + sY11, aY12 + sY12,
          -aY02 + sY02, -aY12 + sY12, mY - sY00 - sY11]
    for p in range(9):
        yfull[p] = yf[p]
    for s in range(2):
        ypacked[s] = jnp.concatenate(
            [c[:, s * 16:(s + 1) * 16] for c in comps], axis=1)


def _node_prep(X9, lin2_W, lin3_W, lin4_W):
    Np = X9.shape[1]
    NB = 1000
    grid = Np // NB
    return pl.pallas_call(
        _prep_body,
        grid=(grid,),
        in_specs=[
            pl.BlockSpec((9, NB, H), lambda i: (0, i, 0)),
            pl.BlockSpec((H, H), lambda i: (0, 0)),
            pl.BlockSpec((H, H), lambda i: (0, 0)),
            pl.BlockSpec((H, H), lambda i: (0, 0)),
        ],
        out_specs=[
            pl.BlockSpec((9, NB, H), lambda i: (0, i, 0)),
            pl.BlockSpec((2, NB, 144), lambda i: (0, i, 0)),
        ],
        out_shape=[
            jax.ShapeDtypeStruct((9, Np, H), jnp.float32),
            jax.ShapeDtypeStruct((2, Np, 144), jnp.float32),
        ],
        interpret=_INTERPRET,
    )(X9, lin2_W.T, lin3_W.T, lin4_W.T)


def _final_body(x, yfull, mp, w5t, w6t, w7t, t0t, t1t, t2t,
                ls0t, ls0b, ls1pt, ls1pb, out):
    xs = [x[p] for p in range(9)]
    yf = [yfull[p] for p in range(9)]
    macc = [jnp.concatenate([mp[0, :, c * 16:(c + 1) * 16],
                             mp[1, :, c * 16:(c + 1) * 16]], axis=1)
            for c in range(9)]
    M = [macc[0] + macc[4], macc[1] + macc[5], macc[2] + macc[6],
         -macc[1] + macc[5], macc[0] + macc[7], macc[3] + macc[8],
         -macc[2] + macc[6], -macc[3] + macc[8], macc[0] - macc[4] - macc[7]]
    Z = _mm(M, _mm(yf, M))
    mZ, za01, za02, za12, zs00, zs01, zs02, zs11, zs12 = _decomp(Z)
    dot = lambda a, wt: jnp.dot(a, wt[...], preferred_element_type=jnp.float32)
    mW = dot(mZ, w7t)
    wa01 = dot(za01, w6t); wa02 = dot(za02, w6t); wa12 = dot(za12, w6t)
    ws00 = dot(zs00, w5t); ws01 = dot(zs01, w5t); ws02 = dot(zs02, w5t)
    ws11 = dot(zs11, w5t); ws12 = dot(zs12, w5t)
    Y2 = [xs[0] + mW + ws00, xs[1] + wa01 + ws01, xs[2] + wa02 + ws02,
          xs[3] - wa01 + ws01, xs[4] + mW + ws11, xs[5] + wa12 + ws12,
          xs[6] - wa02 + ws02, xs[7] - wa12 + ws12, xs[8] + mW - ws00 - ws11]
    tn = sum(p * p for p in Y2) + 1.0
    inv = 1.0 / tn
    Y2 = [p * inv for p in Y2]
    m3, b01, b02, b12, c00, c01, c02, c11, c12 = _decomp(Y2)
    tnI = 3.0 * m3 * m3
    tnA = 2.0 * (b01 * b01 + b02 * b02 + b12 * b12)
    c22 = -c00 - c11
    tnS = (c00 * c00 + c11 * c11 + c22 * c22
           + 2.0 * (c01 * c01 + c02 * c02 + c12 * c12))
    nrm = jnp.concatenate([tnI, tnA, tnS], axis=1)
    nrm = jnp.tanh(dot(nrm, ls0t) + ls0b[...])
    nrm = jnp.tanh(dot(nrm, ls1pt) + ls1pb[...])
    f0 = nrm[:, :H]; f1 = nrm[:, H:2 * H]; f2 = nrm[:, 2 * H:]
    dm = dot(m3, t0t)
    da01 = dot(b01, t1t); da02 = dot(b02, t1t); da12 = dot(b12, t1t)
    ds00 = dot(c00, t2t); ds01 = dot(c01, t2t); ds02 = dot(c02, t2t)
    ds11 = dot(c11, t2t); ds12 = dot(c12, t2t)
    D = [f0 * dm + f2 * ds00, f1 * da01 + f2 * ds01, f1 * da02 + f2 * ds02,
         -f1 * da01 + f2 * ds01, f0 * dm + f2 * ds11, f1 * da12 + f2 * ds12,
         -f1 * da02 + f2 * ds02, -f1 * da12 + f2 * ds12,
         f0 * dm - f2 * (ds00 + ds11)]
    DD = _mm(D, D)
    for p in range(9):
        out[p] = xs[p] + D[p] - 0.5 * DD[p]


def _final(X9, yfull, msgpacked, lin5_W, lin6_W, lin7_W, lt0_W, lt1_W, lt2_W,
           ls0_W, ls0_b, ls1_Wp, ls1_bp):
    Np = X9.shape[1]
    NB = 1000
    grid = Np // NB
    return pl.pallas_call(
        _final_body,
        grid=(grid,),
        in_specs=[
            pl.BlockSpec((9, NB, H), lambda i: (0, i, 0)),
            pl.BlockSpec((9, NB, H), lambda i: (0, i, 0)),
            pl.BlockSpec((2, NB, 144), lambda i: (0, i, 0)),
            pl.BlockSpec((H, H), lambda i: (0, 0)),
            pl.BlockSpec((H, H), lambda i: (0, 0)),
            pl.BlockSpec((H, H), lambda i: (0, 0)),
            pl.BlockSpec((H, H), lambda i: (0, 0)),
            pl.BlockSpec((H, H), lambda i: (0, 0)),
            pl.BlockSpec((H, H), lambda i: (0, 0)),
            pl.BlockSpec((96, 96), lambda i: (0, 0)),
            pl.BlockSpec((1, 96), lambda i: (0, 0)),
            pl.BlockSpec((96, 96), lambda i: (0, 0)),
            pl.BlockSpec((1, 96), lambda i: (0, 0)),
        ],
        out_specs=pl.BlockSpec((9, NB, H), lambda i: (0, i, 0)),
        out_shape=jax.ShapeDtypeStruct((9, Np, H), jnp.float32),
        interpret=_INTERPRET,
    )(X9, yfull, msgpacked, lin5_W.T, lin6_W.T, lin7_W.T,
      lt0_W.T, lt1_W.T, lt2_W.T, ls0_W.T, ls0_b.reshape(1, 96),
      ls1_Wp.T, ls1_bp.reshape(1, 96))


_SC_NC = 2
_SC_NS = 16
_SC_BLK = 160
_SC_SUB = 80


def _sc_message(ypackedf, ea48f, srcN, dst3, npad, Np, Epad):
    from jax.experimental.pallas import tpu_sc as plsc

    EPT = Epad // _SC_NS
    nblk = EPT // _SC_BLK
    nsub = _SC_BLK // _SC_SUB
    stripe = npad // _SC_NS
    mesh = plsc.VectorSubcoreMesh(core_axis_name="c", subcore_axis_name="s")

    @functools.partial(
        pl.kernel, mesh=mesh,
        compiler_params=pltpu.CompilerParams(use_tc_tiling_on_sc=False),
        out_type=jax.ShapeDtypeStruct((2 * npad, 144), jnp.float32),
        scratch_types=[
            pltpu.VMEM((_SC_BLK,), jnp.int32),
            pltpu.VMEM((nsub, 1, _SC_SUB), jnp.int32),
            pltpu.VMEM((nsub, _SC_SUB, 144), jnp.float32),
            pltpu.VMEM((_SC_BLK * 48,), jnp.float32),
            pltpu.VMEM_SHARED((npad, 144), jnp.float32),
            pltpu.SemaphoreType.DMA,
        ],
    )
    def body(yp_hbm, ea_hbm, src_hbm, dst_hbm, out_hbm,
             src_v, dst_v, rows_v, f_v, acc, sem):
        c = lax.axis_index("c")
        s = lax.axis_index("s")

        def zfill(e, carry):
            for j in range(9):
                rows_v[0, e, pl.ds(j * 16, 16)] = jnp.zeros((16,), jnp.float32)
            return carry

        lax.fori_loop(0, _SC_SUB, zfill, 0)
        for j in range(stripe // _SC_SUB):
            pltpu.sync_copy(rows_v.at[0],
                            acc.at[pl.ds(s * stripe + j * _SC_SUB, _SC_SUB)])
        plsc.subcore_barrier()

        def blk(i, carry):
            base = s * EPT + i * _SC_BLK
            pltpu.sync_copy(src_hbm.at[pl.ds(c * Epad + base, _SC_BLK)],
                            src_v)
            pltpu.sync_copy(dst_hbm.at[base // _SC_BLK], dst_v)
            cps = [pltpu.async_copy(yp_hbm.at[src_v.at[pl.ds(j * _SC_SUB,
                                                             _SC_SUB)]],
                                    rows_v.at[j], sem)
                   for j in range(nsub)]
            pltpu.sync_copy(ea_hbm.at[pl.ds((c * Epad + base) * 48,
                                             _SC_BLK * 48)], f_v)
            for cp in cps:
                cp.wait()

            for j8 in range(nsub):
                def edge(e, carry2):
                    fb = (j8 * _SC_SUB + e) * 48
                    wk = [f_v[pl.ds(fb + 16 * k, 16)] for k in range(3)]
                    for j in range(9):
                        rows_v[j8, e, pl.ds(j * 16, 16)] = \
                            rows_v[j8, e, pl.ds(j * 16, 16)] * wk[WMAP[j]]
                    return carry2

                lax.fori_loop(0, _SC_SUB, edge, 0)
            for j in range(nsub):
                pltpu.sync_copy(rows_v.at[j], acc.at[dst_v.at[j, 0]],
                                add=True)
            return carry

        lax.fori_loop(0, nblk, blk, 0)
        plsc.subcore_barrier()
        pltpu.sync_copy(acc.at[pl.ds(s * stripe, stripe)],
                        out_hbm.at[pl.ds(c * npad + s * stripe, stripe)])

    out = body(ypackedf, ea48f, srcN, dst3)
    return jnp.concatenate([out[:Np], out[npad:npad + Np]], axis=0)


def _message_pass(ypacked, ea48, src, dst, Np, Epad):
    srcN = jnp.concatenate([src, src + Np])
    dst3 = dst.reshape(Epad // _SC_BLK, _SC_BLK // _SC_SUB, 1, _SC_SUB)
    npad = ((Np + _SC_BLK * _SC_NS - 1) // (_SC_BLK * _SC_NS)) \
        * _SC_BLK * _SC_NS
    out = _sc_message(ypacked.reshape(2 * Np, 144),
                      ea48.reshape(2 * Epad * 48),
                      srcN, dst3, npad, Np, Epad)
    return out.reshape(2, Np, 144)


def kernel(X, edge_index, edge_weight, edge_attr, lin0_W, lin0_b, lin1_W,
           lin1_b, lin2_W, lin3_W, lin4_W, lin5_W, lin6_W, lin7_W, lt0_W,
           lt1_W, lt2_W, ls0_W, ls0_b, ls1_W, ls1_b):
    Np = X.shape[0]
    E = edge_index.shape[1]
    X9 = jnp.transpose(X.reshape(Np, H, 9), (2, 0, 1))

    unit = _SC_BLK * _SC_NS
    Epad = ((E + unit - 1) // unit) * unit
    pad = Epad - E
    edge_attr_p = jnp.pad(edge_attr, ((0, pad), (0, 0)))
    edge_weight_p = jnp.pad(edge_weight, (0, pad), constant_values=1e6)
    src = jnp.pad(edge_index[0].astype(jnp.int32), (0, pad))
    dst = jnp.pad(edge_index[1].astype(jnp.int32), (0, pad))

    ea48 = _edge_mlp(edge_attr_p, edge_weight_p, lin0_W, lin0_b, lin1_W,
                     lin1_b)
    yfull, ypacked = _node_prep(X9, lin2_W, lin3_W, lin4_W)
    msgpacked = _message_pass(ypacked, ea48, src, dst, Np, Epad)

    perm = np.array([3 * h + k for k in range(3) for h in range(H)])
    out9 = _final(X9, yfull, msgpacked, lin5_W, lin6_W, lin7_W,
                  lt0_W, lt1_W, lt2_W, ls0_W, ls0_b, ls1_W[perm], ls1_b[perm])
    return jnp.transpose(out9, (1, 2, 0)).reshape(Np, H, 3, 3)
